# baseline (speedup 1.0000x reference)
import jax
import jax.numpy as jnp
from jax.experimental import pallas as pl

N, M, D = 10000, 32, 128
NB = 200


def _body(x_ref, o_ref):
    o_ref[...] = jnp.sum(x_ref[...], axis=1)


def kernel(messages):
    x = messages.reshape(N, M, D)
    out = pl.pallas_call(
        _body,
        grid=(N // NB,),
        in_specs=[pl.BlockSpec((NB, M, D), lambda i: (i, 0, 0))],
        out_specs=pl.BlockSpec((NB, D), lambda i: (i, 0)),
        out_shape=jax.ShapeDtypeStruct((N, D), jnp.float32),
    )(x)
    return out.reshape(1, N, D)
